# baseline (speedup 1.0000x reference)
import functools

import jax
import jax.numpy as jnp
from jax import lax
from jax.experimental import pallas as pl
from jax.experimental.pallas import tpu as pltpu
from jax.experimental.pallas import tpu_sc as plsc

N = 50000
E = 800000
D_NODE = 128
D_EDGE = 16
EMB = 32
P = 64
LAM = 1.0

NC, NS = 2, 16
NW = NC * NS
CHUNK = 128
E_PAD = 802816
NROWS = E_PAD // CHUNK
TRASH = N
N_TBL = 50048

CHN = 2048
CHE = 4096

_f32 = jnp.float32
_bf16 = jnp.bfloat16


def _full(spec_shape):
    return pl.BlockSpec(spec_shape, lambda i: (0, 0))


def _node_pre_body(h, ne_w1, ne_b1, ne_w2, ne_b2, wa, wb, h_in, a, b):
    t = jnp.maximum(jnp.dot(h[...], ne_w1[...], preferred_element_type=_f32)
                    + ne_b1[...], 0.0)
    hi = jnp.dot(t, ne_w2[...], preferred_element_type=_f32) + ne_b2[...]
    h_in[...] = hi
    a[...] = jnp.dot(hi, wa[...], preferred_element_type=_f32)
    b[...] = jnp.dot(hi, wb[...], preferred_element_type=_f32)


def _node_pre(h, ne_w1, ne_b1, ne_w2, ne_b2, wa, wb):
    grid = (N_TBL + CHN - 1) // CHN
    return pl.pallas_call(
        _node_pre_body,
        grid=(grid,),
        in_specs=[
            pl.BlockSpec((CHN, D_NODE), lambda i: (i, 0)),
            _full((D_NODE, EMB)), _full((1, EMB)),
            _full((EMB, EMB)), _full((1, EMB)),
            _full((EMB, P)), _full((EMB, P)),
        ],
        out_specs=[
            pl.BlockSpec((CHN, EMB), lambda i: (i, 0)),
            pl.BlockSpec((CHN, P), lambda i: (i, 0)),
            pl.BlockSpec((CHN, P), lambda i: (i, 0)),
        ],
        out_shape=[
            jax.ShapeDtypeStruct((N_TBL, EMB), _f32),
            jax.ShapeDtypeStruct((N_TBL, P), _f32),
            jax.ShapeDtypeStruct((N_TBL, P), _f32),
        ],
    )(h, ne_w1, ne_b1, ne_w2, ne_b2, wa, wb)


def _idx_prep_body(iv, src_o, dst_o):
    i = pl.program_id(0)
    blk = iv[...].reshape(128, 2, 128)
    rr = lax.broadcasted_iota(jnp.int32, (128, 128), 0) + i * 128
    valid = rr < (E // CHUNK)
    src_o[...] = jnp.where(valid, blk[:, 0, :], TRASH)
    dst_o[...] = jnp.where(valid, blk[:, 1, :], TRASH)


def _idx_prep(idx_view):
    return pl.pallas_call(
        _idx_prep_body,
        grid=(NROWS // 128,),
        in_specs=[pl.BlockSpec((256, 128), lambda i: (i, 0))],
        out_specs=[
            pl.BlockSpec((128, 128), lambda i: (i, 0)),
            pl.BlockSpec((128, 128), lambda i: (i, 0)),
        ],
        out_shape=[
            jax.ShapeDtypeStruct((NROWS, CHUNK), jnp.int32),
            jax.ShapeDtypeStruct((NROWS, CHUNK), jnp.int32),
        ],
    )(idx_view)


NROWS_H = NROWS // 4
_WROWS = NROWS_H // NW
_GB = 7
_GROUPS = _WROWS // _GB


def _gather_body(row_base, a_h, b_h, src_h, dst_h, tok_in, s_h, tok_out,
                 idx_s, idx_d, sbuf, sem):
    c = lax.axis_index("c")
    s = lax.axis_index("s")
    wid = s * NC + c
    base = row_base + wid * _WROWS

    def grp(g, carry):
        row0 = base + g * _GB
        pltpu.sync_copy(src_h.at[pl.ds(row0, _GB)], idx_s)
        pltpu.sync_copy(dst_h.at[pl.ds(row0, _GB)], idx_d)
        descs = [
            pltpu.async_copy(a_h.at[idx_s.at[i]],
                             sbuf.at[pl.ds(i * CHUNK, CHUNK)], sem)
            for i in range(_GB)
        ]
        for d in descs:
            d.wait()
        descs = [
            pltpu.async_copy(b_h.at[idx_d.at[i]],
                             sbuf.at[pl.ds(i * CHUNK, CHUNK)], sem, add=True)
            for i in range(_GB)
        ]
        for d in descs:
            d.wait()
        pltpu.sync_copy(sbuf,
                        s_h.at[pl.ds((row0 - row_base) * CHUNK, _GB * CHUNK),
                               pl.ds(0, P)])
        return carry

    lax.fori_loop(0, _GROUPS, grp, 0)


def _gather(a, b, src2, dst2, row_base, tok):
    mesh = plsc.VectorSubcoreMesh(core_axis_name="c", subcore_axis_name="s")
    return pl.kernel(
        functools.partial(_gather_body, row_base),
        out_type=[jax.ShapeDtypeStruct((NROWS_H * CHUNK, 128), _f32),
                  jax.ShapeDtypeStruct((8, 128), _f32)],
        mesh=mesh,
        compiler_params=pltpu.CompilerParams(use_tc_tiling_on_sc=False),
        scratch_types=[
            pltpu.VMEM((_GB, CHUNK), jnp.int32),
            pltpu.VMEM((_GB, CHUNK), jnp.int32),
            pltpu.VMEM((_GB * CHUNK, P), _f32),
            pltpu.SemaphoreType.DMA,
        ],
    )(a, b, src2, dst2, tok)


def _edge_body(et, s, w, ee_w1, ee_b1, w2c, bc, pe_w2, pe_b2,
               ed_w1, ed_b1, ed_w2, ed_b2, e_out, msg_w):
    t1 = jnp.maximum(
        lax.dot_general(et[...], ee_w1[...], (((0,), (0,)), ((), ())),
                        preferred_element_type=_f32) + ee_b1[...], 0.0)
    cc = jnp.dot(t1, w2c[...], preferred_element_type=_f32) + bc[...]
    ey = (lax.broadcasted_iota(jnp.int32, (128, 128), 0)
          == lax.broadcasted_iota(jnp.int32, (128, 128), 1)).astype(_f32)
    wt = lax.dot_general(ey, w[...], (((1,), (1,)), ((), ())),
                         preferred_element_type=_f32)
    w = jnp.concatenate([wt[:, k:k + 1] for k in range(CHE // 128)], axis=0)
    t = jnp.maximum(s[...][:, :P] + cc, 0.0)
    e_new = jnp.dot(t, pe_w2[...], preferred_element_type=_f32) + pe_b2[...]
    msg = e_new * w
    msg_w[...] = jnp.concatenate([msg, jnp.zeros((CHE, P), _f32)], axis=1)
    d1 = jnp.maximum(jnp.dot(e_new, ed_w1[...], preferred_element_type=_f32)
                     + ed_b1[...], 0.0)
    e_dec = jnp.dot(d1, ed_w2[...], preferred_element_type=_f32) + ed_b2[...]
    r = LAM * jnp.sqrt(w)
    nrm = jnp.sqrt(jnp.sum(e_dec * e_dec, axis=1, keepdims=True))
    e_out[...] = e_dec * jnp.minimum(1.0, r / jnp.maximum(nrm, 1e-12))


def _edge_half_body(et, s, w, ee_w1, ee_b1, w2c, bc, pe_w2, pe_b2,
                    ed_w1, ed_b1, ed_w2, ed_b2, e_prev, e_out, msg_w):
    _edge_body(et, s, w, ee_w1, ee_b1, w2c, bc, pe_w2, pe_b2,
               ed_w1, ed_b1, ed_w2, ed_b2, e_out, msg_w)


def _edge_mega(e_t, s_arr, w128, ee_w1, ee_b1, w2c, bc, pe_w2,
               pe_b2, ed_w1, ed_b1, ed_w2, ed_b2, e_prev, half):
    off = half * (NROWS_H * CHUNK // CHE)
    in_specs = [
        pl.BlockSpec((D_EDGE, CHE), lambda i: (0, i + off)),
        pl.BlockSpec((CHE, 128), lambda i: (i, 0)),
        pl.BlockSpec((CHE // 128, 128), lambda i: (i + off, 0)),
        _full((D_EDGE, EMB)), _full((1, EMB)),
        _full((EMB, P)), _full((1, P)),
        _full((P, P)), _full((1, P)),
        _full((P, EMB)), _full((1, EMB)),
        _full((EMB, D_NODE)), _full((1, D_NODE)),
    ]
    args = [e_t, s_arr, w128, ee_w1, ee_b1, w2c, bc, pe_w2, pe_b2,
            ed_w1, ed_b1, ed_w2, ed_b2]
    if e_prev is not None:
        body = _edge_half_body
        in_specs.append(pl.BlockSpec((8, D_NODE), lambda i: (0, 0)))
        args.append(e_prev)
        aliases = {13: 0}
    else:
        body = _edge_body
        aliases = {}
    return pl.pallas_call(
        body,
        grid=(NROWS_H * CHUNK // CHE,),
        in_specs=in_specs,
        out_specs=[
            pl.BlockSpec((CHE, D_NODE), lambda i: (i + off, 0)),
            pl.BlockSpec((CHE, 128), lambda i: (i, 0)),
        ],
        out_shape=[
            jax.ShapeDtypeStruct((E, D_NODE), _f32),
            jax.ShapeDtypeStruct((NROWS_H * CHUNK, 128), _f32),
        ],
        input_output_aliases=aliases,
    )(*args)


_ZCH = 184
_ZN = 17
_SROWS = NROWS_H // NS
_SGB = 7
_SGROUPS = _SROWS // _SGB


def _scatter_body(row_base, msg_h, dst_h, tok_in, agg_h, tok_out, idx, mbuf, acc, sem):
    c = lax.axis_index("c")
    s = lax.axis_index("s")
    zero16 = jnp.zeros((16,), _f32)

    def zrow(r, carry):
        mbuf[r, pl.ds(0, 16)] = zero16
        mbuf[r, pl.ds(16, 16)] = zero16
        return carry

    lax.fori_loop(0, _ZCH, zrow, 0)

    def zcp(g, carry):
        pltpu.sync_copy(mbuf.at[pl.ds(0, _ZCH)],
                        acc.at[pl.ds(s * (_ZN * _ZCH) + g * _ZCH, _ZCH)])
        return carry

    lax.fori_loop(0, _ZN, zcp, 0)
    plsc.subcore_barrier()

    def run(col0):
        def grp(g, carry):
            row0 = s * _SROWS + g * _SGB
            pltpu.sync_copy(dst_h.at[pl.ds(row_base + row0, _SGB)], idx)
            pltpu.sync_copy(msg_h.at[pl.ds(row0 * CHUNK, _SGB * CHUNK),
                                     pl.ds(col0, EMB)], mbuf)
            for i in range(_SGB):
                pltpu.sync_copy(mbuf.at[pl.ds(i * CHUNK, CHUNK)],
                                acc.at[idx.at[i]], add=True)
            return carry

        lax.fori_loop(0, _SGROUPS, grp, 0)

    @pl.when(c == 0)
    def _():
        run(0)

    @pl.when(c == 1)
    def _():
        run(EMB)

    plsc.subcore_barrier()

    def readout(col0):
        def rcp(g, carry):
            r0 = s * (_ZN * _ZCH) + g * _ZCH
            pltpu.sync_copy(acc.at[pl.ds(r0, _ZCH)], mbuf.at[pl.ds(0, _ZCH)])
            pltpu.sync_copy(mbuf.at[pl.ds(0, _ZCH)],
                            agg_h.at[pl.ds(r0, _ZCH), pl.ds(col0, EMB)])
            return carry

        lax.fori_loop(0, _ZN, rcp, 0)

    @pl.when(c == 0)
    def _():
        readout(0)

    @pl.when(c == 1)
    def _():
        readout(EMB)


def _scatter(msg_w, dst2, row_base, tok):
    mesh = plsc.VectorSubcoreMesh(core_axis_name="c", subcore_axis_name="s")
    return pl.kernel(
        functools.partial(_scatter_body, row_base),
        out_type=[jax.ShapeDtypeStruct((N_TBL, 128), _f32),
                  jax.ShapeDtypeStruct((8, 128), _f32)],
        mesh=mesh,
        compiler_params=pltpu.CompilerParams(use_tc_tiling_on_sc=False),
        scratch_types=[
            pltpu.VMEM((_SGB, CHUNK), jnp.int32),
            pltpu.VMEM((_SGB * CHUNK, EMB), _f32),
            pltpu.VMEM_SHARED((N_TBL, EMB), _f32),
            pltpu.SemaphoreType.DMA,
        ],
    )(msg_w, dst2, tok)


def _node_post_body(h_in, agg1, agg2, agg3, agg4, wn1h, wn1al, wn1ar, pn_b1,
                    w2d1, bd1, nd_w2, nd_b2, h_out):
    aggv = (agg1[...] + agg2[...]) + (agg3[...] + agg4[...])
    z = jnp.dot(h_in[...], wn1h[...], preferred_element_type=_f32)
    z += jnp.dot(aggv[:, :EMB], wn1al[...], preferred_element_type=_f32)
    z += jnp.dot(aggv[:, EMB:P], wn1ar[...], preferred_element_type=_f32)
    z = jnp.maximum(z + pn_b1[...], 0.0)
    d = jnp.maximum(jnp.dot(z, w2d1[...], preferred_element_type=_f32)
                    + bd1[...], 0.0)
    h_out[...] = jnp.dot(d, nd_w2[...], preferred_element_type=_f32) + nd_b2[...]


def _node_post(h_in, aggs, wn1h, wn1al, wn1ar, pn_b1, w2d1, bd1,
               nd_w2, nd_b2):
    grid = (N + CHN - 1) // CHN
    return pl.pallas_call(
        _node_post_body,
        grid=(grid,),
        in_specs=[
            pl.BlockSpec((CHN, EMB), lambda i: (i, 0)),
            pl.BlockSpec((CHN, 128), lambda i: (i, 0)),
            pl.BlockSpec((CHN, 128), lambda i: (i, 0)),
            pl.BlockSpec((CHN, 128), lambda i: (i, 0)),
            pl.BlockSpec((CHN, 128), lambda i: (i, 0)),
            _full((EMB, P)), _full((EMB, P)), _full((EMB, P)),
            _full((1, P)), _full((P, EMB)), _full((1, EMB)),
            _full((EMB, D_NODE)), _full((1, D_NODE)),
        ],
        out_specs=pl.BlockSpec((CHN, D_NODE), lambda i: (i, 0)),
        out_shape=jax.ShapeDtypeStruct((N, D_NODE), _f32),
    )(h_in, *aggs, wn1h, wn1al, wn1ar, pn_b1, w2d1, bd1,
      nd_w2, nd_b2)


def kernel(h, e, edge_index, w, x, params):
    p = params

    def r2(v):
        return v.reshape(1, -1)

    wa = p['pe_w1'][0:32] + p['pe_w1'][32:64]
    wb = p['pe_w1'][64:96] + p['pe_w1'][96:128]
    wc = p['pe_w1'][128:160] + p['pe_w1'][160:192]
    wn1h = p['pn_w1'][0:32] + p['pn_w1'][32:64]
    wn1al = p['pn_w1'][64:96]
    wn1ar = p['pn_w1'][96:128]

    idx_view = (edge_index.astype(jnp.int32)
                .reshape(2, E // CHUNK, CHUNK)
                .swapaxes(0, 1)
                .reshape(2 * (E // CHUNK), CHUNK))
    src2, dst2 = _idx_prep(idx_view)
    w128 = jnp.concatenate([w, jnp.zeros((E_PAD - E,), _f32)]
                           ).reshape(NROWS, CHUNK)
    e_t = e.T

    hp = jax.lax.Precision.HIGHEST
    w2c = jnp.dot(p['ee_w2'], wc, precision=hp)
    bc = jnp.dot(r2(p['ee_b2']), wc, precision=hp) + r2(p['pe_b1'])
    w2d1 = jnp.dot(p['pn_w2'], p['nd_w1'], precision=hp)
    bd1 = jnp.dot(r2(p['pn_b2']), p['nd_w1'], precision=hp) + r2(p['nd_b1'])

    h_in, a_tbl, b_tbl = _node_pre(h, p['ne_w1'], r2(p['ne_b1']), p['ne_w2'],
                                   r2(p['ne_b2']), wa, wb)
    ew = (p['ee_w1'], r2(p['ee_b1']), w2c, bc, p['pe_w2'], r2(p['pe_b2']),
          p['ed_w1'], r2(p['ed_b1']), p['ed_w2'], r2(p['ed_b2']))
    tok0 = jnp.zeros((8, 128), _f32)
    s1, t = _gather(a_tbl, b_tbl, src2, dst2, 0, tok0)
    s2, t = _gather(a_tbl, b_tbl, src2, dst2, NROWS_H, t)
    s3, t = _gather(a_tbl, b_tbl, src2, dst2, 2 * NROWS_H, t)
    e_out, msg1 = _edge_mega(e_t, s1, w128, *ew, e_prev=None, half=0)
    agg1, t = _scatter(msg1, dst2, 0, t)
    s4, t = _gather(a_tbl, b_tbl, src2, dst2, 3 * NROWS_H, t)
    e_out, msg2 = _edge_mega(e_t, s2, w128, *ew, e_prev=e_out, half=1)
    agg2, t = _scatter(msg2, dst2, NROWS_H, t)
    e_out, msg3 = _edge_mega(e_t, s3, w128, *ew, e_prev=e_out, half=2)
    agg3, t = _scatter(msg3, dst2, 2 * NROWS_H, t)
    e_out, msg4 = _edge_mega(e_t, s4, w128, *ew, e_prev=e_out, half=3)
    agg4, t = _scatter(msg4, dst2, 3 * NROWS_H, t)
    aggs = [agg1, agg2, agg3, agg4]
    h_out = _node_post(h_in, aggs, wn1h, wn1al, wn1ar, r2(p['pn_b1']),
                       w2d1, bd1, p['nd_w2'], r2(p['nd_b2']))
    return (h_out, e_out)


# speedup vs baseline: 1.0730x; 1.0730x over previous
import functools

import jax
import jax.numpy as jnp
from jax import lax
from jax.experimental import pallas as pl
from jax.experimental.pallas import tpu as pltpu
from jax.experimental.pallas import tpu_sc as plsc

N = 50000
E = 800000
D_NODE = 128
D_EDGE = 16
EMB = 32
P = 64
LAM = 1.0

NC, NS = 2, 16
NW = NC * NS
CHUNK = 128
E_PAD = 802816
NROWS = E_PAD // CHUNK
TRASH = N
N_TBL = 50048

CHN = 2048
CHE = 8192

_f32 = jnp.float32
_bf16 = jnp.bfloat16


def _full(spec_shape):
    return pl.BlockSpec(spec_shape, lambda i: (0, 0))


def _node_pre_body(h, ne_w1, ne_b1, ne_w2, ne_b2, wa, wb, h_in, a, b):
    t = jnp.maximum(jnp.dot(h[...], ne_w1[...], preferred_element_type=_f32)
                    + ne_b1[...], 0.0)
    hi = jnp.dot(t, ne_w2[...], preferred_element_type=_f32) + ne_b2[...]
    h_in[...] = hi
    a[...] = jnp.dot(hi, wa[...], preferred_element_type=_f32)
    b[...] = jnp.dot(hi, wb[...], preferred_element_type=_f32)


def _node_pre(h, ne_w1, ne_b1, ne_w2, ne_b2, wa, wb):
    grid = (N_TBL + CHN - 1) // CHN
    return pl.pallas_call(
        _node_pre_body,
        grid=(grid,),
        in_specs=[
            pl.BlockSpec((CHN, D_NODE), lambda i: (i, 0)),
            _full((D_NODE, EMB)), _full((1, EMB)),
            _full((EMB, EMB)), _full((1, EMB)),
            _full((EMB, P)), _full((EMB, P)),
        ],
        out_specs=[
            pl.BlockSpec((CHN, EMB), lambda i: (i, 0)),
            pl.BlockSpec((CHN, P), lambda i: (i, 0)),
            pl.BlockSpec((CHN, P), lambda i: (i, 0)),
        ],
        out_shape=[
            jax.ShapeDtypeStruct((N_TBL, EMB), _f32),
            jax.ShapeDtypeStruct((N_TBL, P), _f32),
            jax.ShapeDtypeStruct((N_TBL, P), _f32),
        ],
    )(h, ne_w1, ne_b1, ne_w2, ne_b2, wa, wb)


def _idx_prep_body(iv, src_o, dst_o):
    i = pl.program_id(0)
    blk = iv[...].reshape(128, 2, 128)
    rr = lax.broadcasted_iota(jnp.int32, (128, 128), 0) + i * 128
    valid = rr < (E // CHUNK)
    src_o[...] = jnp.where(valid, blk[:, 0, :], TRASH)
    dst_o[...] = jnp.where(valid, blk[:, 1, :], TRASH)


def _idx_prep(idx_view):
    return pl.pallas_call(
        _idx_prep_body,
        grid=(NROWS // 128,),
        in_specs=[pl.BlockSpec((256, 128), lambda i: (i, 0))],
        out_specs=[
            pl.BlockSpec((128, 128), lambda i: (i, 0)),
            pl.BlockSpec((128, 128), lambda i: (i, 0)),
        ],
        out_shape=[
            jax.ShapeDtypeStruct((NROWS, CHUNK), jnp.int32),
            jax.ShapeDtypeStruct((NROWS, CHUNK), jnp.int32),
        ],
    )(idx_view)


NROWS_H = NROWS // 2
_WROWS = NROWS_H // NW
_GB = 7
_GROUPS = _WROWS // _GB


def _gather_body(row_base, a_h, b_h, src_h, dst_h, tok_in, s_h, tok_out,
                 idx_s, idx_d, sbuf, sem):
    c = lax.axis_index("c")
    s = lax.axis_index("s")
    wid = s * NC + c
    base = row_base + wid * _WROWS

    def grp(g, carry):
        row0 = base + g * _GB
        pltpu.sync_copy(src_h.at[pl.ds(row0, _GB)], idx_s)
        pltpu.sync_copy(dst_h.at[pl.ds(row0, _GB)], idx_d)
        descs = [
            pltpu.async_copy(a_h.at[idx_s.at[i]],
                             sbuf.at[pl.ds(i * CHUNK, CHUNK)], sem)
            for i in range(_GB)
        ]
        for d in descs:
            d.wait()
        descs = [
            pltpu.async_copy(b_h.at[idx_d.at[i]],
                             sbuf.at[pl.ds(i * CHUNK, CHUNK)], sem, add=True)
            for i in range(_GB)
        ]
        for d in descs:
            d.wait()
        pltpu.sync_copy(sbuf,
                        s_h.at[pl.ds((row0 - row_base) * CHUNK, _GB * CHUNK),
                               pl.ds(0, P)])
        return carry

    lax.fori_loop(0, _GROUPS, grp, 0)


def _gather(a, b, src2, dst2, row_base, tok):
    mesh = plsc.VectorSubcoreMesh(core_axis_name="c", subcore_axis_name="s")
    return pl.kernel(
        functools.partial(_gather_body, row_base),
        out_type=[jax.ShapeDtypeStruct((NROWS_H * CHUNK, 128), _f32),
                  jax.ShapeDtypeStruct((8, 128), _f32)],
        mesh=mesh,
        compiler_params=pltpu.CompilerParams(use_tc_tiling_on_sc=False),
        scratch_types=[
            pltpu.VMEM((_GB, CHUNK), jnp.int32),
            pltpu.VMEM((_GB, CHUNK), jnp.int32),
            pltpu.VMEM((_GB * CHUNK, P), _f32),
            pltpu.SemaphoreType.DMA,
        ],
    )(a, b, src2, dst2, tok)


def _edge_body(et, s, w, ee_w1, ee_b1, w2c, bc, pe_w2, pe_b2,
               ed_w1, ed_b1, ed_w2, ed_b2, e_out, msg_w):
    t1 = jnp.maximum(
        lax.dot_general(et[...], ee_w1[...], (((0,), (0,)), ((), ())),
                        preferred_element_type=_f32) + ee_b1[...], 0.0)
    cc = jnp.dot(t1, w2c[...], preferred_element_type=_f32) + bc[...]
    ey = (lax.broadcasted_iota(jnp.int32, (128, 128), 0)
          == lax.broadcasted_iota(jnp.int32, (128, 128), 1)).astype(_f32)
    wt = lax.dot_general(ey, w[...], (((1,), (1,)), ((), ())),
                         preferred_element_type=_f32)
    w = jnp.concatenate([wt[:, k:k + 1] for k in range(CHE // 128)], axis=0)
    t = jnp.maximum(s[...][:, :P] + cc, 0.0)
    e_new = jnp.dot(t, pe_w2[...], preferred_element_type=_f32) + pe_b2[...]
    msg = e_new * w
    msg_w[...] = jnp.concatenate([msg, jnp.zeros((CHE, P), _f32)], axis=1)
    d1 = jnp.maximum(jnp.dot(e_new, ed_w1[...], preferred_element_type=_f32)
                     + ed_b1[...], 0.0)
    e_dec = jnp.dot(d1, ed_w2[...], preferred_element_type=_f32) + ed_b2[...]
    r = LAM * jnp.sqrt(w)
    nrm = jnp.sqrt(jnp.sum(e_dec * e_dec, axis=1, keepdims=True))
    e_out[...] = e_dec * jnp.minimum(1.0, r / jnp.maximum(nrm, 1e-12))


def _edge_half_body(et, s, w, ee_w1, ee_b1, w2c, bc, pe_w2, pe_b2,
                    ed_w1, ed_b1, ed_w2, ed_b2, e_prev, e_out, msg_w):
    _edge_body(et, s, w, ee_w1, ee_b1, w2c, bc, pe_w2, pe_b2,
               ed_w1, ed_b1, ed_w2, ed_b2, e_out, msg_w)


def _edge_mega(e_t, s_arr, w128, ee_w1, ee_b1, w2c, bc, pe_w2,
               pe_b2, ed_w1, ed_b1, ed_w2, ed_b2, e_prev, half):
    off = half * (NROWS_H * CHUNK // CHE)
    in_specs = [
        pl.BlockSpec((D_EDGE, CHE), lambda i: (0, i + off)),
        pl.BlockSpec((CHE, 128), lambda i: (i, 0)),
        pl.BlockSpec((CHE // 128, 128), lambda i: (i + off, 0)),
        _full((D_EDGE, EMB)), _full((1, EMB)),
        _full((EMB, P)), _full((1, P)),
        _full((P, P)), _full((1, P)),
        _full((P, EMB)), _full((1, EMB)),
        _full((EMB, D_NODE)), _full((1, D_NODE)),
    ]
    args = [e_t, s_arr, w128, ee_w1, ee_b1, w2c, bc, pe_w2, pe_b2,
            ed_w1, ed_b1, ed_w2, ed_b2]
    if e_prev is not None:
        body = _edge_half_body
        in_specs.append(pl.BlockSpec((8, D_NODE), lambda i: (0, 0)))
        args.append(e_prev)
        aliases = {13: 0}
    else:
        body = _edge_body
        aliases = {}
    return pl.pallas_call(
        body,
        grid=(NROWS_H * CHUNK // CHE,),
        in_specs=in_specs,
        out_specs=[
            pl.BlockSpec((CHE, D_NODE), lambda i: (i + off, 0)),
            pl.BlockSpec((CHE, 128), lambda i: (i, 0)),
        ],
        out_shape=[
            jax.ShapeDtypeStruct((E, D_NODE), _f32),
            jax.ShapeDtypeStruct((NROWS_H * CHUNK, 128), _f32),
        ],
        input_output_aliases=aliases,
    )(*args)


_ZCH = 184
_ZN = 17
_SROWS = NROWS_H // NS
_SGB = 7
_SGROUPS = _SROWS // _SGB


def _scatter_body(row_base, msg_h, dst_h, tok_in, agg_h, tok_out, idx, mbuf, acc, sem):
    c = lax.axis_index("c")
    s = lax.axis_index("s")
    zero16 = jnp.zeros((16,), _f32)

    def zrow(r, carry):
        mbuf[r, pl.ds(0, 16)] = zero16
        mbuf[r, pl.ds(16, 16)] = zero16
        return carry

    lax.fori_loop(0, _ZCH, zrow, 0)

    def zcp(g, carry):
        pltpu.sync_copy(mbuf.at[pl.ds(0, _ZCH)],
                        acc.at[pl.ds(s * (_ZN * _ZCH) + g * _ZCH, _ZCH)])
        return carry

    lax.fori_loop(0, _ZN, zcp, 0)
    plsc.subcore_barrier()

    def run(col0):
        def grp(g, carry):
            row0 = s * _SROWS + g * _SGB
            pltpu.sync_copy(dst_h.at[pl.ds(row_base + row0, _SGB)], idx)
            pltpu.sync_copy(msg_h.at[pl.ds(row0 * CHUNK, _SGB * CHUNK),
                                     pl.ds(col0, EMB)], mbuf)
            for i in range(_SGB):
                pltpu.sync_copy(mbuf.at[pl.ds(i * CHUNK, CHUNK)],
                                acc.at[idx.at[i]], add=True)
            return carry

        lax.fori_loop(0, _SGROUPS, grp, 0)

    @pl.when(c == 0)
    def _():
        run(0)

    @pl.when(c == 1)
    def _():
        run(EMB)

    plsc.subcore_barrier()

    def readout(col0):
        def rcp(g, carry):
            r0 = s * (_ZN * _ZCH) + g * _ZCH
            pltpu.sync_copy(acc.at[pl.ds(r0, _ZCH)], mbuf.at[pl.ds(0, _ZCH)])
            pltpu.sync_copy(mbuf.at[pl.ds(0, _ZCH)],
                            agg_h.at[pl.ds(r0, _ZCH), pl.ds(col0, EMB)])
            return carry

        lax.fori_loop(0, _ZN, rcp, 0)

    @pl.when(c == 0)
    def _():
        readout(0)

    @pl.when(c == 1)
    def _():
        readout(EMB)


def _scatter(msg_w, dst2, row_base, tok):
    mesh = plsc.VectorSubcoreMesh(core_axis_name="c", subcore_axis_name="s")
    return pl.kernel(
        functools.partial(_scatter_body, row_base),
        out_type=[jax.ShapeDtypeStruct((N_TBL, 128), _f32),
                  jax.ShapeDtypeStruct((8, 128), _f32)],
        mesh=mesh,
        compiler_params=pltpu.CompilerParams(use_tc_tiling_on_sc=False),
        scratch_types=[
            pltpu.VMEM((_SGB, CHUNK), jnp.int32),
            pltpu.VMEM((_SGB * CHUNK, EMB), _f32),
            pltpu.VMEM_SHARED((N_TBL, EMB), _f32),
            pltpu.SemaphoreType.DMA,
        ],
    )(msg_w, dst2, tok)


def _node_post_body(h_in, agg1, agg2, wn1h, wn1al, wn1ar, pn_b1,
                    w2d1, bd1, nd_w2, nd_b2, h_out):
    aggv = agg1[...] + agg2[...]
    z = jnp.dot(h_in[...], wn1h[...], preferred_element_type=_f32)
    z += jnp.dot(aggv[:, :EMB], wn1al[...], preferred_element_type=_f32)
    z += jnp.dot(aggv[:, EMB:P], wn1ar[...], preferred_element_type=_f32)
    z = jnp.maximum(z + pn_b1[...], 0.0)
    d = jnp.maximum(jnp.dot(z, w2d1[...], preferred_element_type=_f32)
                    + bd1[...], 0.0)
    h_out[...] = jnp.dot(d, nd_w2[...], preferred_element_type=_f32) + nd_b2[...]


def _node_post(h_in, aggs, wn1h, wn1al, wn1ar, pn_b1, w2d1, bd1,
               nd_w2, nd_b2):
    grid = (N + CHN - 1) // CHN
    return pl.pallas_call(
        _node_post_body,
        grid=(grid,),
        in_specs=[
            pl.BlockSpec((CHN, EMB), lambda i: (i, 0)),
            pl.BlockSpec((CHN, 128), lambda i: (i, 0)),
            pl.BlockSpec((CHN, 128), lambda i: (i, 0)),
            _full((EMB, P)), _full((EMB, P)), _full((EMB, P)),
            _full((1, P)), _full((P, EMB)), _full((1, EMB)),
            _full((EMB, D_NODE)), _full((1, D_NODE)),
        ],
        out_specs=pl.BlockSpec((CHN, D_NODE), lambda i: (i, 0)),
        out_shape=jax.ShapeDtypeStruct((N, D_NODE), _f32),
    )(h_in, *aggs, wn1h, wn1al, wn1ar, pn_b1, w2d1, bd1,
      nd_w2, nd_b2)


def kernel(h, e, edge_index, w, x, params):
    p = params

    def r2(v):
        return v.reshape(1, -1)

    wa = p['pe_w1'][0:32] + p['pe_w1'][32:64]
    wb = p['pe_w1'][64:96] + p['pe_w1'][96:128]
    wc = p['pe_w1'][128:160] + p['pe_w1'][160:192]
    wn1h = p['pn_w1'][0:32] + p['pn_w1'][32:64]
    wn1al = p['pn_w1'][64:96]
    wn1ar = p['pn_w1'][96:128]

    idx_view = (edge_index.astype(jnp.int32)
                .reshape(2, E // CHUNK, CHUNK)
                .swapaxes(0, 1)
                .reshape(2 * (E // CHUNK), CHUNK))
    src2, dst2 = _idx_prep(idx_view)
    w128 = jnp.concatenate([w, jnp.zeros((E_PAD - E,), _f32)]
                           ).reshape(NROWS, CHUNK)
    e_t = e.T

    hp = jax.lax.Precision.HIGHEST
    w2c = jnp.dot(p['ee_w2'], wc, precision=hp)
    bc = jnp.dot(r2(p['ee_b2']), wc, precision=hp) + r2(p['pe_b1'])
    w2d1 = jnp.dot(p['pn_w2'], p['nd_w1'], precision=hp)
    bd1 = jnp.dot(r2(p['pn_b2']), p['nd_w1'], precision=hp) + r2(p['nd_b1'])

    h_in, a_tbl, b_tbl = _node_pre(h, p['ne_w1'], r2(p['ne_b1']), p['ne_w2'],
                                   r2(p['ne_b2']), wa, wb)
    ew = (p['ee_w1'], r2(p['ee_b1']), w2c, bc, p['pe_w2'], r2(p['pe_b2']),
          p['ed_w1'], r2(p['ed_b1']), p['ed_w2'], r2(p['ed_b2']))
    tok0 = jnp.zeros((8, 128), _f32)
    s1, t = _gather(a_tbl, b_tbl, src2, dst2, 0, tok0)
    s2, t = _gather(a_tbl, b_tbl, src2, dst2, NROWS_H, t)
    e_out, msg1 = _edge_mega(e_t, s1, w128, *ew, e_prev=None, half=0)
    agg1, t = _scatter(msg1, dst2, 0, t)
    e_out, msg2 = _edge_mega(e_t, s2, w128, *ew, e_prev=e_out, half=1)
    agg2, t = _scatter(msg2, dst2, NROWS_H, t)
    h_out = _node_post(h_in, [agg1, agg2], wn1h, wn1al, wn1ar, r2(p['pn_b1']),
                       w2d1, bd1, p['nd_w2'], r2(p['nd_b2']))
    return (h_out, e_out)
